# Initial kernel scaffold; baseline (speedup 1.0000x reference)
import jax
import jax.numpy as jnp
from jax.experimental import pallas as pl


def kernel(y_pred, y_true):
    raise NotImplementedError("write your pallas kernel here")


# speedup vs baseline: 1.6956x; 1.6956x over previous
import functools

import jax
import jax.numpy as jnp
from jax import lax
from jax.experimental import pallas as pl
from jax.experimental.pallas import tpu as pltpu

B = 16384
C = 1000
K = 1638
ROWS = 256
GRID = B // ROWS


def _tc_body(x_ref, yt_ref, p0_ref, palt_ref, diff_ref, m2_ref):
    x = x_ref[...]
    yt = yt_ref[...]
    m = jnp.max(x, axis=1, keepdims=True)
    e = jnp.exp(x - m)
    s = jnp.sum(e, axis=1, keepdims=True)
    col = lax.broadcasted_iota(jnp.int32, (ROWS, C), 1)
    is0 = yt == 0
    c_alt = jnp.where(is0, 1, yt)
    e_alt = jnp.sum(jnp.where(col == c_alt, e, 0.0), axis=1, keepdims=True)
    val = 2.0 * e - jnp.where(col == yt, s, 0.0)
    val = jnp.where(col == 0, jnp.float32(-3.0e38), val)
    m2 = jnp.max(val, axis=1, keepdims=True) / s
    p0 = e[:, 0:1] / s
    diff = m2 - 2.0 * p0 + is0.astype(jnp.float32)
    p0_ref[...] = p0
    palt_ref[...] = e_alt / s
    diff_ref[...] = diff
    m2_ref[...] = m2


def _tc_stage(y_pred, yt2d):
    out = jax.ShapeDtypeStruct((B, 1), jnp.float32)
    row_spec = pl.BlockSpec((ROWS, 1), lambda i: (i, 0))
    return pl.pallas_call(
        _tc_body,
        grid=(GRID,),
        in_specs=[
            pl.BlockSpec((ROWS, C), lambda i: (i, 0)),
            pl.BlockSpec((ROWS, 1), lambda i: (i, 0)),
        ],
        out_specs=[row_spec, row_spec, row_spec, row_spec],
        out_shape=[out, out, out, out],
        compiler_params=pltpu.CompilerParams(
            dimension_semantics=("arbitrary",),
        ),
    )(y_pred, yt2d)


def kernel(y_pred, y_true):
    yt2d = y_true.reshape(B, 1)
    p0, palt, diff, m2 = _tc_stage(y_pred, yt2d)
    p0 = p0.reshape(B)
    palt = palt.reshape(B)
    diff = diff.reshape(B)
    m2 = m2.reshape(B)
    is0 = y_true == 0
    n0 = jnp.sum(is0.astype(jnp.int32))
    rank0 = jnp.cumsum(is0.astype(jnp.int32)) - is0.astype(jnp.int32)
    rank1 = jnp.arange(B, dtype=jnp.int32) - rank0
    chosen_t = jnp.where(is0, rank0 < K, rank1 < (K - n0))
    p_ctr = jnp.where(chosen_t, p0, palt)
    sd = jnp.sort(diff)
    sum_k = jnp.sum(sd[:K])
    return (jnp.sum(p_ctr) - jnp.sum(m2) + sum_k) / B
